# Initial kernel scaffold; baseline (speedup 1.0000x reference)
import jax
import jax.numpy as jnp
from jax.experimental import pallas as pl


def kernel(h, msgs, inject, identity, edge_index, w_conn, hebbian, Ws1, bs1, Ws2, bs2, Wm1, bm1, Wm2, bm2, Wd1, bd1, Wd2, bd2):
    raise NotImplementedError("write your pallas kernel here")


# speedup vs baseline: 4.9807x; 4.9807x over previous
import functools

import jax
import jax.numpy as jnp
from jax import lax
from jax.experimental import pallas as pl
from jax.experimental.pallas import tpu as pltpu
from jax.experimental.pallas import tpu_sc as plsc

BS = 2
N = 10000
K = 32
D = 128
D_ID = 32
E = N * K
CH = 128
NSUB = 16
E_PAD = ((E + NSUB * CH - 1) // (NSUB * CH)) * (NSUB * CH)
EPT = E_PAD // NSUB
NCH = EPT // CH
N_PAD = 10240
ROWS_PER_TEC = N_PAD // NSUB


def _recv_sc(msgs_flat, src_pad, tgt_pad, w_pad):
    mesh = plsc.VectorSubcoreMesh(core_axis_name="c", subcore_axis_name="s")

    @functools.partial(
        pl.kernel,
        mesh=mesh,
        out_type=jax.ShapeDtypeStruct((BS * N_PAD, D), jnp.float32),
        scratch_types=[
            pltpu.VMEM((1, CH), jnp.int32),
            pltpu.VMEM((1, CH), jnp.int32),
            pltpu.VMEM((1, CH), jnp.float32),
            pltpu.VMEM((CH, D), jnp.float32),
            pltpu.VMEM_SHARED((N_PAD, D), jnp.float32),
            pltpu.SemaphoreType.DMA,
        ],
    )
    def k(msgs_hbm, src_hbm, tgt_hbm, w_hbm, out_hbm,
          src_v, tgt_v, w_v, rows_v, acc_sh, sem):
        c = lax.axis_index("c")
        s = lax.axis_index("s")

        zero16 = jnp.zeros((16,), jnp.float32)

        def zrow(i, carry):
            for g in range(8):
                rows_v[i, pl.ds(g * 16, 16)] = zero16
            return carry

        lax.fori_loop(0, CH, zrow, 0)
        base = s * ROWS_PER_TEC
        for j in range(ROWS_PER_TEC // CH):
            pltpu.sync_copy(rows_v, acc_sh.at[pl.ds(base + j * CH, CH)])
        rem = ROWS_PER_TEC % CH
        if rem:
            pltpu.sync_copy(rows_v.at[pl.ds(0, rem)],
                            acc_sh.at[pl.ds(base + (ROWS_PER_TEC // CH) * CH, rem)])
        plsc.subcore_barrier()

        coff = c * N

        def chunk(ci, carry):
            eb = s * EPT + ci * CH
            pltpu.sync_copy(src_hbm.at[pl.ds(eb, CH)], src_v.at[0])
            pltpu.sync_copy(tgt_hbm.at[pl.ds(eb, CH)], tgt_v.at[0])
            pltpu.sync_copy(w_hbm.at[pl.ds(c * E_PAD + eb, CH)], w_v.at[0])
            for g in range(8):
                sl = pl.ds(g * 16, 16)
                src_v[0, sl] = src_v[0, sl] + coff
                wv = w_v[0, sl]
                w_v[0, sl] = 1.0 / (1.0 + jnp.exp(-wv))
            pltpu.async_copy(msgs_hbm.at[src_v.at[0]], rows_v, sem).wait()

            def scale(gi, icarry):
                wg = w_v[0, pl.ds(gi * 16, 16)]
                for j in range(16):
                    w = wg[j]
                    r = gi * 16 + j
                    for g in range(8):
                        sl = pl.ds(g * 16, 16)
                        rows_v[r, sl] = rows_v[r, sl] * w
                return icarry

            lax.fori_loop(0, CH // 16, scale, 0)
            pltpu.sync_copy(rows_v, acc_sh.at[tgt_v.at[0]], add=True)
            return carry

        lax.fori_loop(0, NCH, chunk, 0)
        plsc.subcore_barrier()
        pltpu.sync_copy(acc_sh.at[pl.ds(base, ROWS_PER_TEC)],
                        out_hbm.at[pl.ds(c * N_PAD + base, ROWS_PER_TEC)])

    return k(msgs_flat, src_pad, tgt_pad, w_pad)


BLK = 2000


def _mlp_body(hebb_ref, h_ref, id_ref, recv_ref, inj_ref,
              Wd1_ref, bd1_ref, Wd2_ref, bd2_ref,
              Ws1_ref, bs1_ref, Ws2_ref, bs2_ref,
              Wm1_ref, bm1_ref, Wm2_ref, bm2_ref,
              hnew_ref, msgs_ref, wconn_ref, decay_ref, idnew_ref):
    dot = functools.partial(jnp.dot, preferred_element_type=jnp.float32,
                            precision=lax.Precision.HIGHEST)
    Wd1 = Wd1_ref[...]
    Wd2 = Wd2_ref[...]
    Ws1 = Ws1_ref[...]
    Ws2 = Ws2_ref[...]
    Wm1 = Wm1_ref[...]
    Wm2 = Wm2_ref[...]
    bd1 = bd1_ref[...]
    bd2 = bd2_ref[...]
    bs1 = bs1_ref[...]
    bs2 = bs2_ref[...]
    bm1 = bm1_ref[...]
    bm2 = bm2_ref[...]
    idb = id_ref[...]

    mod_outs = []
    for b in range(BS):
        hb = hebb_ref[b]
        hh = h_ref[b]
        rc = recv_ref[b]
        ij = inj_ref[b]
        t = (dot(hb, Wd1[0:K]) + dot(hh, Wd1[K:K + D])
             + dot(idb, Wd1[K + D:K + D + D_ID])
             + dot(rc, Wd1[K + D + D_ID:K + 2 * D + D_ID])
             + dot(ij, Wd1[K + 2 * D + D_ID:K + 3 * D + D_ID]) + bd1)
        mod_h = t * jax.nn.sigmoid(t)
        mod_out = dot(mod_h, Wd2) + bd2
        mod_outs.append(mod_out)
        wconn_ref[b] = mod_out[:, :K]

    id_new = idb + 0.5 * (mod_outs[0][:, K + 1:] + mod_outs[1][:, K + 1:])
    idnew_ref[...] = id_new

    for b in range(BS):
        hh = h_ref[b]
        rc = recv_ref[b]
        ij = inj_ref[b]
        decay = jax.nn.sigmoid(mod_outs[b][:, K:K + 1])
        decay_ref[b] = decay
        t = (dot(rc, Ws1[0:D]) + dot(ij, Ws1[D:2 * D]) + dot(hh, Ws1[2 * D:3 * D])
             + dot(id_new, Ws1[3 * D:3 * D + D_ID]) + bs1)
        update = jnp.tanh(dot(t * jax.nn.sigmoid(t), Ws2) + bs2)
        h_new = decay * hh + (1.0 - decay) * update
        hnew_ref[b] = h_new
        t = dot(h_new, Wm1[0:D]) + dot(id_new, Wm1[D:D + D_ID]) + bm1
        msgs_ref[b] = jnp.tanh(dot(t * jax.nn.sigmoid(t), Wm2) + bm2)


def _mlps_tc(hebbian, h, identity, received, inject,
             Wd1, bd1, Wd2, bd2, Ws1, bs1, Ws2, bs2, Wm1, bm1, Wm2, bm2,
             interpret=False):
    nblk = N // BLK
    bspec = lambda shp, imap: pl.BlockSpec(shp, imap)
    node3 = lambda w: pl.BlockSpec((BS, BLK, w), lambda i: (0, i, 0))
    full = lambda a: pl.BlockSpec(a.shape, lambda i: tuple(0 for _ in a.shape))
    out_shapes = (
        jax.ShapeDtypeStruct((BS, N, D), jnp.float32),
        jax.ShapeDtypeStruct((BS, N, D), jnp.float32),
        jax.ShapeDtypeStruct((BS, N, K), jnp.float32),
        jax.ShapeDtypeStruct((BS, N, 1), jnp.float32),
        jax.ShapeDtypeStruct((N, D_ID), jnp.float32),
    )
    out_specs = (
        node3(D), node3(D), node3(K), node3(1),
        pl.BlockSpec((BLK, D_ID), lambda i: (i, 0)),
    )
    in_specs = [
        node3(K),
        node3(D),
        pl.BlockSpec((BLK, D_ID), lambda i: (i, 0)),
        node3(D),
        node3(D),
        full(Wd1), full(bd1), full(Wd2), full(bd2),
        full(Ws1), full(bs1), full(Ws2), full(bs2),
        full(Wm1), full(bm1), full(Wm2), full(bm2),
    ]
    return pl.pallas_call(
        _mlp_body,
        grid=(nblk,),
        in_specs=in_specs,
        out_specs=out_specs,
        out_shape=out_shapes,
        interpret=interpret,
    )(hebbian, h, identity, received, inject,
      Wd1, bd1, Wd2, bd2, Ws1, bs1, Ws2, bs2, Wm1, bm1, Wm2, bm2)


def kernel(h, msgs, inject, identity, edge_index, w_conn, hebbian,
           Ws1, bs1, Ws2, bs2, Wm1, bm1, Wm2, bm2, Wd1, bd1, Wd2, bd2):
    src = edge_index[0].astype(jnp.int32)
    tgt = edge_index[1].astype(jnp.int32)
    pad = E_PAD - E
    src_pad = jnp.pad(src, (0, pad))
    tgt_pad = jnp.pad(tgt, (0, pad))
    w_pad = jnp.pad(w_conn.reshape(BS, E), ((0, 0), (0, pad)),
                    constant_values=-1e9).reshape(-1)
    msgs_flat = msgs.reshape(BS * N, D)
    received = _recv_sc(msgs_flat, src_pad, tgt_pad, w_pad)
    received = received.reshape(BS, N_PAD, D)[:, :N]
    return _mlps_tc(hebbian, h, identity, received, inject,
                    Wd1, bd1, Wd2, bd2, Ws1, bs1, Ws2, bs2, Wm1, bm1, Wm2, bm2)
